# baseline (speedup 1.0000x reference)
import functools

import jax
import jax.numpy as jnp
from jax import lax
from jax.experimental import pallas as pl
from jax.experimental.pallas import tpu as pltpu
from jax.experimental.pallas import tpu_sc as plsc


_LANES = 16
_CH = 128


def _pad_nodes(n):
    return ((n + 1 + 127) // 128) * 128


def _mm1_body(x_ref, w1_ref, o_ref):
    n = x_ref.shape[0]
    np_ = o_ref.shape[0]
    o_ref[pl.ds(0, n), :] = jnp.dot(x_ref[...], w1_ref[...],
                                    preferred_element_type=jnp.float32)
    o_ref[pl.ds(n, np_ - n), :] = jnp.zeros((np_ - n, o_ref.shape[1]),
                                            jnp.float32)


def _mid_body(u_ref, s_ref, b1_ref, w2_ref, b2_ref, o_ref):
    z = u_ref[...] + s_ref[0] + s_ref[1] + b1_ref[...]
    t = jnp.maximum(z, 0.0)
    h = jnp.dot(t, w2_ref[...], preferred_element_type=jnp.float32) + b2_ref[...]
    o_ref[...] = jnp.maximum(h, 0.0)


def _post_body(h_ref, s_ref, w3_ref, b3_ref, w4_ref, b4_ref, o_ref):
    n = o_ref.shape[0]
    z2 = h_ref[...] + s_ref[0] + s_ref[1]
    h2 = jnp.maximum(
        jnp.dot(z2, w3_ref[...], preferred_element_type=jnp.float32) + b3_ref[...],
        0.0)
    o = jnp.dot(h2, w4_ref[...], preferred_element_type=jnp.float32) + b4_ref[...]
    o = o[:n, :]
    m = jnp.max(o, axis=1, keepdims=True)
    e = jnp.exp(o - m)
    lse = jnp.log(jnp.sum(e, axis=1, keepdims=True))
    o_ref[...] = o - m - lse


@functools.lru_cache(maxsize=None)
def _make_segsum(np_, n_chunks, hidden, pw0_frac_16):
    assert hidden == _LANES
    rows_per_tile = np_ // 16
    assert rows_per_tile % 8 == 0

    G = 4
    NB = 2
    per_w_sum = n_chunks // 16
    assert per_w_sum * 16 == n_chunks
    pw = [per_w_sum * pw0_frac_16 // 16, 0]
    pw[0] -= pw[0] % (G * NB)
    pw[1] = per_w_sum - pw[0]
    assert pw[1] % (G * NB) == 0 and pw[0] >= G * NB and pw[1] >= G * NB
    pw_max = max(pw)

    mesh = plsc.VectorSubcoreMesh(core_axis_name="c", subcore_axis_name="s")

    @functools.partial(
        pl.kernel, mesh=mesh,
        out_type=jax.ShapeDtypeStruct((2, np_, hidden), jnp.float32),
        scratch_types=[
            pltpu.VMEM((pw_max, _CH), jnp.int32),
            pltpu.VMEM((pw_max, _CH), jnp.int32),
            pltpu.VMEM((NB * G * _CH, hidden), jnp.float32),
            pltpu.VMEM_SHARED((np_, hidden), jnp.float32),
            pltpu.SemaphoreType.DMA,
        ],
        compiler_params=pltpu.CompilerParams(use_tc_tiling_on_sc=False),
    )
    def segsum(table_hbm, src_hbm, dst_hbm, zeros_hbm, out_hbm,
               src_v, dst_v, rows_v, acc_sh, gsem):
        c = lax.axis_index("c")
        s = lax.axis_index("s")

        row0 = s * rows_per_tile
        pltpu.sync_copy(zeros_hbm.at[pl.ds(row0, rows_per_tile)],
                        acc_sh.at[pl.ds(row0, rows_per_tile)])

        @pl.when(c == 0)
        def _():
            chunk0 = s * pw[0]
            pltpu.sync_copy(src_hbm.at[pl.ds(chunk0, pw[0])],
                            src_v.at[pl.ds(0, pw[0])])
            pltpu.sync_copy(dst_hbm.at[pl.ds(chunk0, pw[0])],
                            dst_v.at[pl.ds(0, pw[0])])

        @pl.when(c == 1)
        def _():
            chunk0 = 16 * pw[0] + s * pw[1]
            pltpu.sync_copy(src_hbm.at[pl.ds(chunk0, pw[1])],
                            src_v.at[pl.ds(0, pw[1])])
            pltpu.sync_copy(dst_hbm.at[pl.ds(chunk0, pw[1])],
                            dst_v.at[pl.ds(0, pw[1])])

        ngroups = jnp.where(c == 0, pw[0] // G, pw[1] // G)

        plsc.subcore_barrier()

        def gather_dma(i, b, k):
            return pltpu.make_async_copy(
                table_hbm.at[src_v.at[i * G + k]],
                rows_v.at[pl.ds((b * G + k) * _CH, _CH)],
                gsem)

        for b in range(NB):
            for k in range(G):
                gather_dma(jnp.int32(b), b, k).start()

        def outer(i0, carry):
            for b in range(NB):
                i = i0 * NB + b
                for k in range(G):
                    gather_dma(i, b, k).wait()
                for k in range(G):
                    pltpu.sync_copy(rows_v.at[pl.ds((b * G + k) * _CH, _CH)],
                                    acc_sh.at[dst_v.at[i * G + k]], add=True)
                nxt = i + NB

                @pl.when(nxt < ngroups)
                def _():
                    for k in range(G):
                        gather_dma(nxt, b, k).start()
            return carry
        lax.fori_loop(0, ngroups // NB, outer, 0, unroll=False)

        plsc.subcore_barrier()

        pltpu.sync_copy(acc_sh.at[pl.ds(row0, rows_per_tile)],
                        out_hbm.at[c, pl.ds(row0, rows_per_tile)])

    return segsum


def kernel(x, edge_index, W1, b1, W2, b2, W3, b3, W4, b4):
    n = x.shape[0]
    h_dim = W1.shape[1]
    n_edges = edge_index.shape[1]
    np_ = _pad_nodes(n)

    n_chunks = -(-n_edges // _CH)
    n_chunks = -(-n_chunks // 256) * 256
    e_pad = n_chunks * _CH - n_edges
    src = jnp.concatenate(
        [edge_index[0], jnp.full((e_pad,), n, edge_index.dtype)])
    dst = jnp.concatenate(
        [edge_index[1], jnp.full((e_pad,), n, edge_index.dtype)])
    src2d = src.reshape(n_chunks, _CH)
    dst2d = dst.reshape(n_chunks, _CH)
    zeros = jnp.zeros((np_, h_dim), jnp.float32)
    b1r, b2r, b3r, b4r = (b.reshape(1, -1) for b in (b1, b2, b3, b4))

    segsum = _make_segsum(np_, n_chunks, h_dim, 5)

    u = pl.pallas_call(
        _mm1_body,
        out_shape=jax.ShapeDtypeStruct((np_, h_dim), jnp.float32),
    )(x, W1)

    s1 = segsum(u, src2d, dst2d, zeros)

    h = pl.pallas_call(
        _mid_body,
        out_shape=jax.ShapeDtypeStruct((np_, h_dim), jnp.float32),
    )(u, s1, b1r, W2, b2r)

    s2 = segsum(h, src2d, dst2d, zeros)

    out = pl.pallas_call(
        _post_body,
        out_shape=jax.ShapeDtypeStruct((n, W4.shape[1]), jnp.float32),
    )(h, s2, W3, b3r, W4, b4r)

    return out


# speedup vs baseline: 1.1173x; 1.1173x over previous
import functools

import jax
import jax.numpy as jnp
from jax import lax
from jax.experimental import pallas as pl
from jax.experimental.pallas import tpu as pltpu
from jax.experimental.pallas import tpu_sc as plsc


_LANES = 16
_CH = 128


def _pad_nodes(n):
    return ((n + 1 + 127) // 128) * 128


def _mm1_body(x_ref, w1_ref, o_ref):
    n = x_ref.shape[0]
    np_ = o_ref.shape[0]
    o_ref[pl.ds(0, n), :] = jnp.dot(x_ref[...], w1_ref[...],
                                    preferred_element_type=jnp.float32)
    o_ref[pl.ds(n, np_ - n), :] = jnp.zeros((np_ - n, o_ref.shape[1]),
                                            jnp.float32)


def _mid_body(u_ref, s_ref, b1_ref, w2_ref, b2_ref, o_ref):
    z = u_ref[...] + s_ref[0] + s_ref[1] + b1_ref[...]
    t = jnp.maximum(z, 0.0)
    h = jnp.dot(t, w2_ref[...], preferred_element_type=jnp.float32) + b2_ref[...]
    o_ref[...] = jnp.maximum(h, 0.0)


def _post_body(h_ref, s_ref, w3_ref, b3_ref, w4_ref, b4_ref, o_ref):
    n = o_ref.shape[0]
    z2 = h_ref[...] + s_ref[0] + s_ref[1]
    h2 = jnp.maximum(
        jnp.dot(z2, w3_ref[...], preferred_element_type=jnp.float32) + b3_ref[...],
        0.0)
    o = jnp.dot(h2, w4_ref[...], preferred_element_type=jnp.float32) + b4_ref[...]
    o = o[:n, :]
    m = jnp.max(o, axis=1, keepdims=True)
    e = jnp.exp(o - m)
    lse = jnp.log(jnp.sum(e, axis=1, keepdims=True))
    o_ref[...] = o - m - lse


@functools.lru_cache(maxsize=None)
def _make_segsum(np_, n_chunks, hidden, pw0):
    assert hidden == _LANES
    rows_per_tile = np_ // 16
    assert rows_per_tile % 8 == 0

    G = 4
    NB = 2
    per_w_sum = n_chunks // 16
    assert per_w_sum * 16 == n_chunks
    pw = [pw0, per_w_sum - pw0]
    assert pw[0] % (G * NB) == 0 and pw[1] % (G * NB) == 0
    assert pw[0] >= G * NB and pw[1] >= G * NB
    pw_max = max(pw)

    mesh = plsc.VectorSubcoreMesh(core_axis_name="c", subcore_axis_name="s")

    @functools.partial(
        pl.kernel, mesh=mesh,
        out_type=jax.ShapeDtypeStruct((2, np_, hidden), jnp.float32),
        scratch_types=[
            pltpu.VMEM((pw_max, _CH), jnp.int32),
            pltpu.VMEM((pw_max, _CH), jnp.int32),
            pltpu.VMEM((NB * G * _CH, hidden), jnp.float32),
            pltpu.VMEM_SHARED((np_, hidden), jnp.float32),
            pltpu.SemaphoreType.DMA,
        ],
        compiler_params=pltpu.CompilerParams(use_tc_tiling_on_sc=False),
    )
    def segsum(table_hbm, src_hbm, dst_hbm, zeros_hbm, out_hbm,
               src_v, dst_v, rows_v, acc_sh, gsem):
        c = lax.axis_index("c")
        s = lax.axis_index("s")

        row0 = s * rows_per_tile
        pltpu.sync_copy(zeros_hbm.at[pl.ds(row0, rows_per_tile)],
                        acc_sh.at[pl.ds(row0, rows_per_tile)])

        @pl.when(c == 0)
        def _():
            chunk0 = s * pw[0]
            pltpu.sync_copy(src_hbm.at[pl.ds(chunk0, pw[0])],
                            src_v.at[pl.ds(0, pw[0])])
            pltpu.sync_copy(dst_hbm.at[pl.ds(chunk0, pw[0])],
                            dst_v.at[pl.ds(0, pw[0])])

        @pl.when(c == 1)
        def _():
            chunk0 = 16 * pw[0] + s * pw[1]
            pltpu.sync_copy(src_hbm.at[pl.ds(chunk0, pw[1])],
                            src_v.at[pl.ds(0, pw[1])])
            pltpu.sync_copy(dst_hbm.at[pl.ds(chunk0, pw[1])],
                            dst_v.at[pl.ds(0, pw[1])])

        ngroups = jnp.where(c == 0, pw[0] // G, pw[1] // G)

        plsc.subcore_barrier()

        def gather_dma(i, b, k):
            return pltpu.make_async_copy(
                table_hbm.at[src_v.at[i * G + k]],
                rows_v.at[pl.ds((b * G + k) * _CH, _CH)],
                gsem)

        for b in range(NB):
            for k in range(G):
                gather_dma(jnp.int32(b), b, k).start()

        def outer(i0, carry):
            for b in range(NB):
                i = i0 * NB + b
                for k in range(G):
                    gather_dma(i, b, k).wait()
                for k in range(G):
                    pltpu.sync_copy(rows_v.at[pl.ds((b * G + k) * _CH, _CH)],
                                    acc_sh.at[dst_v.at[i * G + k]], add=True)
                nxt = i + NB

                @pl.when(nxt < ngroups)
                def _():
                    for k in range(G):
                        gather_dma(nxt, b, k).start()
            return carry
        lax.fori_loop(0, ngroups // NB, outer, 0, unroll=False)

        plsc.subcore_barrier()

        pltpu.sync_copy(acc_sh.at[pl.ds(row0, rows_per_tile)],
                        out_hbm.at[c, pl.ds(row0, rows_per_tile)])

    return segsum


def kernel(x, edge_index, W1, b1, W2, b2, W3, b3, W4, b4):
    n = x.shape[0]
    h_dim = W1.shape[1]
    n_edges = edge_index.shape[1]
    np_ = _pad_nodes(n)

    n_chunks = -(-n_edges // _CH)
    n_chunks = -(-n_chunks // 256) * 256
    e_pad = n_chunks * _CH - n_edges
    src = jnp.concatenate(
        [edge_index[0], jnp.full((e_pad,), n, edge_index.dtype)])
    dst = jnp.concatenate(
        [edge_index[1], jnp.full((e_pad,), n, edge_index.dtype)])
    src2d = src.reshape(n_chunks, _CH)
    dst2d = dst.reshape(n_chunks, _CH)
    zeros = jnp.zeros((np_, h_dim), jnp.float32)
    b1r, b2r, b3r, b4r = (b.reshape(1, -1) for b in (b1, b2, b3, b4))

    segsum = _make_segsum(np_, n_chunks, h_dim, 112)

    u = pl.pallas_call(
        _mm1_body,
        out_shape=jax.ShapeDtypeStruct((np_, h_dim), jnp.float32),
    )(x, W1)

    s1 = segsum(u, src2d, dst2d, zeros)

    h = pl.pallas_call(
        _mid_body,
        out_shape=jax.ShapeDtypeStruct((np_, h_dim), jnp.float32),
    )(u, s1, b1r, W2, b2r)

    s2 = segsum(h, src2d, dst2d, zeros)

    out = pl.pallas_call(
        _post_body,
        out_shape=jax.ShapeDtypeStruct((n, W4.shape[1]), jnp.float32),
    )(h, s2, W3, b3r, W4, b4r)

    return out
